# baseline (speedup 1.0000x reference)
import functools
import math

import jax
import jax.numpy as jnp
from jax import lax
from jax.experimental import pallas as pl
from jax.experimental.pallas import tpu as pltpu
from jax.experimental.pallas import tpu_sc as plsc

DIM = 64
SCALE = math.sqrt(DIM)
NC = 2
NS = 16
NW = NC * NS


def _build(A, Bdim):
    AT = A // 128
    BT = Bdim // 8
    assert AT == NW and Bdim % 8 == 0

    mesh = plsc.VectorSubcoreMesh(core_axis_name="c", subcore_axis_name="s")

    @functools.partial(
        pl.kernel,
        mesh=mesh,
        out_type=jax.ShapeDtypeStruct((Bdim, DIM // 8, AT, 1024), jnp.float32),
        scratch_types=[
            pltpu.VMEM((BT, 1024), jnp.int32),
            pltpu.VMEM((128, DIM), jnp.float32),
            pltpu.VMEM((128, DIM), jnp.float32),
            pltpu.VMEM((DIM // 8, 1024), jnp.float32),
            pltpu.VMEM((DIM // 8, 1024), jnp.float32),
            pltpu.SemaphoreType.DMA,
            pltpu.SemaphoreType.DMA,
            pltpu.SemaphoreType.DMA,
            pltpu.SemaphoreType.DMA,
        ],
        compiler_params=pltpu.CompilerParams(
            use_tc_tiling_on_sc=False, needs_layout_passes=False),
    )
    def k(tok_hbm, table_hbm, out_hbm, tok_v, rows0, rows1, ov0, ov1,
          g0, g1, o0, o1):
        at = lax.axis_index("s") * NC + lax.axis_index("c")
        pltpu.sync_copy(tok_hbm.at[:, at], tok_v)

        iota = lax.iota(jnp.int32, 16)

        def fire_gather(b, rows_v, sem):
            idx_row = tok_v.at[b >> 3, pl.ds((b & 7) * 128, 128)]
            pltpu.async_copy(table_hbm.at[idx_row], rows_v, sem)

        def drain_gather(rows_v, sem):
            pltpu.make_async_copy(
                table_hbm.at[pl.ds(0, 128)], rows_v, sem).wait()

        def transpose_scale(rows_v, out_v):
            @plsc.parallel_loop(0, 8, unroll=8)
            def _(js):
                for jt in range(DIM // 8):
                    jv = jnp.full((16,), jt * 8, jnp.int32) + js
                    for c in range(8):
                        out_v[jt, pl.ds(js * 128 + c * 16, 16)] = (
                            plsc.load_gather(rows_v, [iota + c * 16, jv])
                            * SCALE)

        def fire_out(b, out_v, sem):
            pltpu.async_copy(out_v, out_hbm.at[b, :, at], sem)

        def drain_out(out_v, sem):
            pltpu.make_async_copy(out_v, out_hbm.at[0, :, 0], sem).wait()

        bufs = ((rows0, ov0, g0, o0), (rows1, ov1, g1, o1))

        def chunk_step(c, par):
            rows_b, ov_b, g_b, o_b = bufs[par]
            rows_n, _, g_n, _ = bufs[1 - par]
            drain_gather(rows_b, g_b)

            @pl.when(c + 1 < Bdim)
            def _():
                fire_gather(c + 1, rows_n, g_n)

            @pl.when(c >= 2)
            def _():
                drain_out(ov_b, o_b)

            transpose_scale(rows_b, ov_b)
            fire_out(c, ov_b, o_b)

        fire_gather(jnp.int32(0), rows0, g0)

        def pair(j, carry):
            chunk_step(2 * j, 0)
            chunk_step(2 * j + 1, 1)
            return carry

        lax.fori_loop(0, Bdim // 2, pair, 0)

        drain_out(ov0, o0)
        drain_out(ov1, o1)

    return k


def kernel(tokens, embedding_weight):
    A, Bdim = tokens.shape
    tok4 = (tokens.astype(jnp.int32).T
            .reshape(Bdim // 8, 8, A // 128, 128)
            .transpose(0, 2, 1, 3)
            .reshape(Bdim // 8, A // 128, 1024))
    out = _build(A, Bdim)(tok4, embedding_weight)
    return (out.reshape(Bdim, DIM // 8, A // 128, 8, 128)
            .transpose(2, 4, 0, 1, 3)
            .reshape(A, Bdim, DIM))


# speedup vs baseline: 1.8885x; 1.8885x over previous
import functools
import math

import jax
import jax.numpy as jnp
from jax import lax
from jax.experimental import pallas as pl
from jax.experimental.pallas import tpu as pltpu
from jax.experimental.pallas import tpu_sc as plsc

DIM = 64
SCALE = math.sqrt(DIM)
NC = 2
NS = 16
NW = NC * NS
PADW = 133


def _build(A, Bdim):
    AT = A // 128
    BT = Bdim // 8
    assert AT == NW and Bdim % 8 == 0

    mesh = plsc.VectorSubcoreMesh(core_axis_name="c", subcore_axis_name="s")

    @functools.partial(
        pl.kernel,
        mesh=mesh,
        out_type=jax.ShapeDtypeStruct((Bdim, DIM // 8, AT, 8, 128),
                                      jnp.float32),
        scratch_types=[
            pltpu.VMEM((BT, 1024), jnp.int32),
            pltpu.VMEM((128, DIM), jnp.float32),
            pltpu.VMEM((128, DIM), jnp.float32),
            pltpu.VMEM((DIM // 8, 8, PADW), jnp.float32),
            pltpu.VMEM((DIM // 8, 8, PADW), jnp.float32),
            pltpu.SemaphoreType.DMA,
            pltpu.SemaphoreType.DMA,
            pltpu.SemaphoreType.DMA,
            pltpu.SemaphoreType.DMA,
        ],
        compiler_params=pltpu.CompilerParams(
            use_tc_tiling_on_sc=False, needs_layout_passes=False),
    )
    def k(tok_hbm, table_hbm, out_hbm, tok_v, rows0, rows1, ov0, ov1,
          g0, g1, o0, o1):
        at = lax.axis_index("s") * NC + lax.axis_index("c")
        pltpu.sync_copy(tok_hbm.at[:, at], tok_v)

        iota = lax.iota(jnp.int32, 16)
        js_idx = jnp.bitwise_and(iota, 7)
        jt_idx = [jnp.right_shift(iota, 3) + 2 * c for c in range(DIM // 16)]

        def fire_gather(b, rows_v, sem):
            idx_row = tok_v.at[b >> 3, pl.ds((b & 7) * 128, 128)]
            pltpu.async_copy(table_hbm.at[idx_row], rows_v, sem)

        def drain_gather(rows_v, sem):
            pltpu.make_async_copy(
                table_hbm.at[pl.ds(0, 128)], rows_v, sem).wait()

        def transpose_scale(rows_v, out_v):
            @plsc.parallel_loop(0, 128, unroll=8)
            def _(a):
                av = jnp.broadcast_to(a, (16,))
                for c in range(DIM // 16):
                    x = rows_v[a, pl.ds(c * 16, 16)] * SCALE
                    plsc.store_scatter(out_v, [jt_idx[c], js_idx, av], x)

        def fire_out(b, out_v, sem):
            pltpu.async_copy(
                out_v.at[:, :, pl.ds(0, 128)], out_hbm.at[b, :, at], sem)

        def drain_out(out_v, sem):
            pltpu.make_async_copy(
                out_v.at[:, :, pl.ds(0, 128)], out_hbm.at[0, :, 0],
                sem).wait()

        bufs = ((rows0, ov0, g0, o0), (rows1, ov1, g1, o1))

        def chunk_step(c, par):
            rows_b, ov_b, g_b, o_b = bufs[par]
            rows_n, _, g_n, _ = bufs[1 - par]
            drain_gather(rows_b, g_b)

            @pl.when(c + 1 < Bdim)
            def _():
                fire_gather(c + 1, rows_n, g_n)

            @pl.when(c >= 2)
            def _():
                drain_out(ov_b, o_b)

            transpose_scale(rows_b, ov_b)
            fire_out(c, ov_b, o_b)

        fire_gather(jnp.int32(0), rows0, g0)

        def pair(j, carry):
            chunk_step(2 * j, 0)
            chunk_step(2 * j + 1, 1)
            return carry

        lax.fori_loop(0, Bdim // 2, pair, 0)

        drain_out(ov0, o0)
        drain_out(ov1, o1)

    return k


def kernel(tokens, embedding_weight):
    A, Bdim = tokens.shape
    tok4 = (tokens.astype(jnp.int32).T
            .reshape(Bdim // 8, 8, A // 128, 128)
            .transpose(0, 2, 1, 3)
            .reshape(Bdim // 8, A // 128, 1024))
    out = _build(A, Bdim)(tok4, embedding_weight)
    return (out.transpose(2, 4, 0, 1, 3)
            .reshape(A, Bdim, DIM))
